# baseline (speedup 1.0000x reference)
import jax
import jax.numpy as jnp
from jax import lax
from jax.experimental import pallas as pl
from jax.experimental.pallas import tpu as pltpu
from jax.experimental.pallas import tpu_sc as plsc

N = 10000
E = 320000
D = 128
G = 8

NC = 2
NS = 16
NW = NC * NS

CH = 125
NCH = 80
HF = NCH // 2
EPT = NCH * CH

NPAD = 10240
RPT = NPAD // NS


def _sc_scatter_kernel(x_hbm, src_hbm, dst_hbm,
                       s_out, cnt_out,
                       src_v, dst_v, rows0_v, rows1_v, ones_v, zcnt_v,
                       sem, gsem0, gsem1, ssem0, ssem1, csem,
                       s_sh, cnt_sh):
    c = lax.axis_index("c")
    s = lax.axis_index("s")
    g = c * NS + s

    pltpu.async_copy(src_hbm.at[g, pl.ds(0, HF)], src_v, sem)
    pltpu.async_copy(dst_hbm.at[g, pl.ds(0, HF)], dst_v, sem)

    zv = jnp.zeros((16,), jnp.float32)
    ov = jnp.ones((16,), jnp.float32)

    @pl.loop(0, CH)
    def _(r):
        for q in range(D // 16):
            rows0_v[r, pl.ds(q * 16, 16)] = zv

    @pl.loop(0, RPT // 16)
    def _(r):
        zcnt_v[pl.ds(r * 16, 16)] = zv

    for q in range(128 // 16):
        ones_v[pl.ds(q * 16, 16)] = ov

    @pl.loop(0, RPT // 80)
    def _(r):
        pltpu.sync_copy(rows0_v.at[pl.ds(0, 80)],
                        s_sh.at[pl.ds(s * RPT + r * 80, 80)])
    pltpu.sync_copy(zcnt_v, cnt_sh.at[pl.ds(s * RPT, RPT)])

    pltpu.make_async_copy(src_hbm.at[g, pl.ds(0, HF)], src_v, sem).wait()
    pltpu.make_async_copy(dst_hbm.at[g, pl.ds(0, HF)], dst_v, sem).wait()
    plsc.subcore_barrier()

    for p in (0, 1):
        if p == 1:
            pltpu.sync_copy(src_hbm.at[g, pl.ds(HF, HF)], src_v)
            pltpu.sync_copy(dst_hbm.at[g, pl.ds(HF, HF)], dst_v)
        pltpu.async_copy(x_hbm.at[src_v.at[0]], rows0_v, gsem0)
        pltpu.async_copy(x_hbm.at[src_v.at[1]], rows1_v, gsem1)

        @pl.loop(0, HF, step=2)
        def _(j):
            pltpu.make_async_copy(x_hbm.at[src_v.at[j]], rows0_v, gsem0).wait()
            pltpu.async_copy(rows0_v, s_sh.at[dst_v.at[j]], ssem0, add=True)
            pltpu.async_copy(ones_v.at[pl.ds(0, CH)], cnt_sh.at[dst_v.at[j]], csem, add=True)

            pltpu.make_async_copy(x_hbm.at[src_v.at[j + 1]], rows1_v, gsem1).wait()
            pltpu.async_copy(rows1_v, s_sh.at[dst_v.at[j + 1]], ssem1, add=True)
            pltpu.async_copy(ones_v.at[pl.ds(0, CH)], cnt_sh.at[dst_v.at[j + 1]], csem, add=True)

            pltpu.make_async_copy(rows0_v, s_sh.at[dst_v.at[j]], ssem0).wait()

            @pl.when(j + 2 < HF)
            def _():
                pltpu.async_copy(x_hbm.at[src_v.at[j + 2]], rows0_v, gsem0)

            pltpu.make_async_copy(rows1_v, s_sh.at[dst_v.at[j + 1]], ssem1).wait()

            @pl.when(j + 3 < HF)
            def _():
                pltpu.async_copy(x_hbm.at[src_v.at[j + 3]], rows1_v, gsem1)

        @pl.loop(0, HF)
        def _(j):
            pltpu.make_async_copy(
                ones_v.at[pl.ds(0, CH)], cnt_sh.at[dst_v.at[j]], csem).wait()

    plsc.subcore_barrier()

    pltpu.sync_copy(s_sh.at[pl.ds(s * RPT, RPT)], s_out.at[c, pl.ds(s * RPT, RPT)])
    pltpu.sync_copy(cnt_sh.at[pl.ds(s * RPT, RPT)], cnt_out.at[c, pl.ds(s * RPT, RPT)])


def _sc_aggregate(x, src3, dst3):
    mesh = plsc.VectorSubcoreMesh(core_axis_name="c", subcore_axis_name="s")
    kern = pl.kernel(
        _sc_scatter_kernel,
        out_type=(
            jax.ShapeDtypeStruct((NC, NPAD, D), jnp.float32),
            jax.ShapeDtypeStruct((NC, NPAD), jnp.float32),
        ),
        mesh=mesh,
        scratch_types=[
            pltpu.VMEM((HF, CH), jnp.int32),
            pltpu.VMEM((HF, CH), jnp.int32),
            pltpu.VMEM((CH, D), jnp.float32),
            pltpu.VMEM((CH, D), jnp.float32),
            pltpu.VMEM((128,), jnp.float32),
            pltpu.VMEM((RPT,), jnp.float32),
            pltpu.SemaphoreType.DMA,
            pltpu.SemaphoreType.DMA,
            pltpu.SemaphoreType.DMA,
            pltpu.SemaphoreType.DMA,
            pltpu.SemaphoreType.DMA,
            pltpu.SemaphoreType.DMA,
            pltpu.VMEM_SHARED((NPAD, D), jnp.float32),
            pltpu.VMEM_SHARED((NPAD,), jnp.float32),
        ],
    )
    return kern(x, src3, dst3)


def _tc_kernel(s0_ref, s1_ref, c0_ref, c1_ref, x_ref, wlT_ref, wrT_ref,
               bl_ref, batch_ref, gw_ref, gb_ref, gms_ref, o_ref):
    s = (s0_ref[:N, :].astype(jnp.float32)
         + s1_ref[:N, :].astype(jnp.float32))
    cnt = c0_ref[:, :N] + c1_ref[:, :N]
    x = x_ref[...]
    inv_cnt = jnp.transpose(1.0 / jnp.maximum(cnt, 1.0))
    aggr = s * inv_cnt
    h = (jnp.dot(aggr, wlT_ref[...], preferred_element_type=jnp.float32)
         + jnp.dot(x, wrT_ref[...], preferred_element_type=jnp.float32)
         + bl_ref[...])
    f = 0.5 * h * (1.0 + lax.erf(h * 0.7071067811865476))

    b = batch_ref[...]
    gid = lax.broadcasted_iota(jnp.int32, (G, N), 0)
    oh = (gid == b).astype(jnp.float32)
    gcnt = jnp.maximum(jnp.sum(oh, axis=1, keepdims=True), 1.0)
    sums = lax.dot_general(oh, f, (((1,), (0,)), ((), ())),
                           preferred_element_type=jnp.float32)
    sqs = lax.dot_general(oh, f * f, (((1,), (0,)), ((), ())),
                          preferred_element_type=jnp.float32)
    gmean = sums / gcnt
    a = gms_ref[...]
    gvar = sqs / gcnt - (2.0 * a - a * a) * gmean * gmean
    m = lax.dot_general(oh, gmean, (((0,), (0,)), ((), ())),
                        preferred_element_type=jnp.float32)
    v = lax.dot_general(oh, gvar, (((0,), (0,)), ((), ())),
                        preferred_element_type=jnp.float32)
    out = (f - a * m) * lax.rsqrt(v + 1e-5)
    o_ref[...] = gw_ref[...] * out + gb_ref[...] + x


def kernel(x, edge_index, batch, W_l, b_l, W_r, gn_weight, gn_bias, gn_mean_scale):
    src3 = edge_index[0].reshape(NW, NCH, CH)
    dst3 = edge_index[1].reshape(NW, NCH, CH)

    s_part, cnt_part = _sc_aggregate(x, src3, dst3)

    out = pl.pallas_call(
        _tc_kernel,
        out_shape=jax.ShapeDtypeStruct((N, D), jnp.float32),
    )(
        s_part[0], s_part[1],
        cnt_part[0].reshape(1, NPAD), cnt_part[1].reshape(1, NPAD),
        x, W_l.T, W_r.T, b_l.reshape(1, D), batch.reshape(1, N),
        gn_weight.reshape(1, D), gn_bias.reshape(1, D),
        gn_mean_scale.reshape(1, D),
    )
    return out


# speedup vs baseline: 1.1920x; 1.1920x over previous
import jax
import jax.numpy as jnp
from jax import lax
from jax.experimental import pallas as pl
from jax.experimental.pallas import tpu as pltpu
from jax.experimental.pallas import tpu_sc as plsc

N = 10000
E = 320000
D = 128
G = 8

NC = 2
NS = 16
NW = NC * NS

CH = 125
NCH = 80
HF = NCH // 2
EPT = NCH * CH

NPAD = 10240
RPT = NPAD // NS


def _sc_scatter_kernel(x_hbm, src_hbm, dst_hbm,
                       s_out, cnt_out,
                       src_v, dst_v, rows0_v, rows1_v, ones_v, zcnt_v,
                       sem, gsem0, gsem1, ssem0, ssem1, csem,
                       s_sh, cnt_sh):
    c = lax.axis_index("c")
    s = lax.axis_index("s")
    g = c * NS + s

    pltpu.async_copy(src_hbm.at[g, pl.ds(0, HF)], src_v, sem)
    pltpu.async_copy(dst_hbm.at[g, pl.ds(0, HF)], dst_v, sem)

    zv = jnp.zeros((16,), jnp.float32)
    ov = jnp.ones((16,), jnp.float32)

    @pl.loop(0, CH)
    def _(r):
        for q in range(D // 16):
            rows0_v[r, pl.ds(q * 16, 16)] = zv

    @pl.loop(0, RPT // 16)
    def _(r):
        zcnt_v[pl.ds(r * 16, 16)] = zv

    for q in range(128 // 16):
        ones_v[pl.ds(q * 16, 16)] = ov

    @pl.loop(0, RPT // 80)
    def _(r):
        pltpu.async_copy(rows0_v.at[pl.ds(0, 80)],
                         s_sh.at[pl.ds(s * RPT + r * 80, 80)], csem)
    pltpu.async_copy(zcnt_v, cnt_sh.at[pl.ds(s * RPT, RPT)], csem)

    @pl.loop(0, RPT // 80)
    def _(r):
        pltpu.make_async_copy(rows0_v.at[pl.ds(0, 80)],
                              s_sh.at[pl.ds(s * RPT + r * 80, 80)], csem).wait()
    pltpu.make_async_copy(zcnt_v, cnt_sh.at[pl.ds(s * RPT, RPT)], csem).wait()

    pltpu.make_async_copy(src_hbm.at[g, pl.ds(0, HF)], src_v, sem).wait()
    pltpu.make_async_copy(dst_hbm.at[g, pl.ds(0, HF)], dst_v, sem).wait()
    plsc.subcore_barrier()

    for p in (0, 1):
        if p == 1:
            pltpu.sync_copy(src_hbm.at[g, pl.ds(HF, HF)], src_v)
            pltpu.sync_copy(dst_hbm.at[g, pl.ds(HF, HF)], dst_v)
        pltpu.async_copy(x_hbm.at[src_v.at[0]], rows0_v, gsem0)
        pltpu.async_copy(x_hbm.at[src_v.at[1]], rows1_v, gsem1)

        @pl.loop(0, HF, step=2)
        def _(j):
            pltpu.make_async_copy(x_hbm.at[src_v.at[j]], rows0_v, gsem0).wait()
            pltpu.sync_copy(rows0_v, s_sh.at[dst_v.at[j]], add=True)
            pltpu.async_copy(ones_v.at[pl.ds(0, CH)], cnt_sh.at[dst_v.at[j]], csem, add=True)

            @pl.when(j + 2 < HF)
            def _():
                pltpu.async_copy(x_hbm.at[src_v.at[j + 2]], rows0_v, gsem0)

            pltpu.make_async_copy(x_hbm.at[src_v.at[j + 1]], rows1_v, gsem1).wait()
            pltpu.sync_copy(rows1_v, s_sh.at[dst_v.at[j + 1]], add=True)
            pltpu.async_copy(ones_v.at[pl.ds(0, CH)], cnt_sh.at[dst_v.at[j + 1]], csem, add=True)

            @pl.when(j + 3 < HF)
            def _():
                pltpu.async_copy(x_hbm.at[src_v.at[j + 3]], rows1_v, gsem1)

        @pl.loop(0, HF)
        def _(j):
            pltpu.make_async_copy(
                ones_v.at[pl.ds(0, CH)], cnt_sh.at[dst_v.at[j]], csem).wait()

    plsc.subcore_barrier()

    pltpu.async_copy(s_sh.at[pl.ds(s * RPT, RPT)],
                     s_out.at[c, pl.ds(s * RPT, RPT)], gsem0)
    pltpu.async_copy(cnt_sh.at[pl.ds(s * RPT, RPT)],
                     cnt_out.at[c, pl.ds(s * RPT, RPT)], gsem1)
    pltpu.make_async_copy(s_sh.at[pl.ds(s * RPT, RPT)],
                          s_out.at[c, pl.ds(s * RPT, RPT)], gsem0).wait()
    pltpu.make_async_copy(cnt_sh.at[pl.ds(s * RPT, RPT)],
                          cnt_out.at[c, pl.ds(s * RPT, RPT)], gsem1).wait()


def _sc_aggregate(x, src3, dst3):
    mesh = plsc.VectorSubcoreMesh(core_axis_name="c", subcore_axis_name="s")
    kern = pl.kernel(
        _sc_scatter_kernel,
        out_type=(
            jax.ShapeDtypeStruct((NC, NPAD, D), jnp.float32),
            jax.ShapeDtypeStruct((NC, NPAD), jnp.float32),
        ),
        mesh=mesh,
        scratch_types=[
            pltpu.VMEM((HF, CH), jnp.int32),
            pltpu.VMEM((HF, CH), jnp.int32),
            pltpu.VMEM((CH, D), jnp.float32),
            pltpu.VMEM((CH, D), jnp.float32),
            pltpu.VMEM((128,), jnp.float32),
            pltpu.VMEM((RPT,), jnp.float32),
            pltpu.SemaphoreType.DMA,
            pltpu.SemaphoreType.DMA,
            pltpu.SemaphoreType.DMA,
            pltpu.SemaphoreType.DMA,
            pltpu.SemaphoreType.DMA,
            pltpu.SemaphoreType.DMA,
            pltpu.VMEM_SHARED((NPAD, D), jnp.float32),
            pltpu.VMEM_SHARED((NPAD,), jnp.float32),
        ],
    )
    return kern(x, src3, dst3)


def _tc_kernel(s_ref, c_ref, x_ref, wlT_ref, wrT_ref,
               bl_ref, batch_ref, gw_ref, gb_ref, gms_ref, o_ref):
    s = s_ref[0, :N, :] + s_ref[1, :N, :]
    cnt = c_ref[0:1, :N] + c_ref[1:2, :N]
    x = x_ref[...]
    inv_cnt = jnp.transpose(1.0 / jnp.maximum(cnt, 1.0))
    aggr = s * inv_cnt
    h = (jnp.dot(aggr, wlT_ref[...], preferred_element_type=jnp.float32)
         + jnp.dot(x, wrT_ref[...], preferred_element_type=jnp.float32)
         + bl_ref[...])
    f = 0.5 * h * (1.0 + lax.erf(h * 0.7071067811865476))

    b = batch_ref[...]
    gid = lax.broadcasted_iota(jnp.int32, (G, N), 0)
    oh = (gid == b).astype(jnp.float32)
    gcnt = jnp.maximum(jnp.sum(oh, axis=1, keepdims=True), 1.0)
    sums = lax.dot_general(oh, f, (((1,), (0,)), ((), ())),
                           preferred_element_type=jnp.float32)
    sqs = lax.dot_general(oh, f * f, (((1,), (0,)), ((), ())),
                          preferred_element_type=jnp.float32)
    gmean = sums / gcnt
    a = gms_ref[...]
    gvar = sqs / gcnt - (2.0 * a - a * a) * gmean * gmean
    m = lax.dot_general(oh, gmean, (((0,), (0,)), ((), ())),
                        preferred_element_type=jnp.float32)
    v = lax.dot_general(oh, gvar, (((0,), (0,)), ((), ())),
                        preferred_element_type=jnp.float32)
    out = (f - a * m) * lax.rsqrt(v + 1e-5)
    o_ref[...] = gw_ref[...] * out + gb_ref[...] + x


def kernel(x, edge_index, batch, W_l, b_l, W_r, gn_weight, gn_bias, gn_mean_scale):
    src3 = edge_index[0].reshape(NW, NCH, CH)
    dst3 = edge_index[1].reshape(NW, NCH, CH)

    s_part, cnt_part = _sc_aggregate(x, src3, dst3)

    out = pl.pallas_call(
        _tc_kernel,
        out_shape=jax.ShapeDtypeStruct((N, D), jnp.float32),
    )(
        s_part, cnt_part,
        x, W_l.T, W_r.T, b_l.reshape(1, D), batch.reshape(1, N),
        gn_weight.reshape(1, D), gn_bias.reshape(1, D),
        gn_mean_scale.reshape(1, D),
    )
    return out
